# Initial kernel scaffold; baseline (speedup 1.0000x reference)
import jax
import jax.numpy as jnp
from jax.experimental import pallas as pl


def kernel(patches, forward_indexes, backward_indexes):
    raise NotImplementedError("write your pallas kernel here")


# speedup vs baseline: 47.1984x; 47.1984x over previous
import functools

import jax
import jax.numpy as jnp
from jax import lax
from jax.experimental import pallas as pl
from jax.experimental.pallas import tpu as pltpu
from jax.experimental.pallas import tpu_sc as plsc

T, B, C = 576, 64, 768
REMAIN = T // 4
N_OUT = REMAIN * B
NW = 32
ROWS_PER_W = N_OUT // NW
CH = 48
NCHUNK = ROWS_PER_W // CH
L = 16


def _body(fwd_hbm, patches_hbm, out_hbm, fwdv, idxv, buf0, buf1, sem0, sem1):
    wid = lax.axis_index("s") * 2 + lax.axis_index("c")
    base = wid * ROWS_PER_W

    pltpu.sync_copy(fwd_hbm.at[pl.ds(base, ROWS_PER_W)], fwdv)

    lanes = lax.iota(jnp.int32, L)
    for j in range(NCHUNK):
        for i in range(CH // L):
            p0 = base + j * CH + i * L
            off = lax.rem(p0, B)
            v = fwdv[pl.ds(j * CH + i * L, L)]
            idxv[j, pl.ds(i * L, L)] = v * B + off + lanes

    bufs = (buf0, buf1)
    sems = (sem0, sem1)
    copies = [None] * NCHUNK
    copies[0] = pltpu.async_copy(patches_hbm.at[idxv.at[0]], bufs[0], sems[0])
    for j in range(NCHUNK):
        if j + 1 < NCHUNK:
            copies[j + 1] = pltpu.async_copy(
                patches_hbm.at[idxv.at[j + 1]], bufs[(j + 1) % 2], sems[(j + 1) % 2]
            )
        copies[j].wait()
        pltpu.sync_copy(bufs[j % 2], out_hbm.at[pl.ds(base + j * CH, CH)])


@jax.jit
def _gather(fwd_flat, patches_flat):
    mesh = plsc.VectorSubcoreMesh(core_axis_name="c", subcore_axis_name="s")
    return pl.kernel(
        _body,
        out_type=jax.ShapeDtypeStruct((N_OUT, C), jnp.float32),
        mesh=mesh,
        scratch_types=[
            pltpu.VMEM((ROWS_PER_W,), jnp.int32),
            pltpu.VMEM((NCHUNK, CH), jnp.int32),
            pltpu.VMEM((CH, C), jnp.float32),
            pltpu.VMEM((CH, C), jnp.float32),
            pltpu.SemaphoreType.DMA,
            pltpu.SemaphoreType.DMA,
        ],
    )(fwd_flat, patches_flat)


def kernel(patches, forward_indexes, backward_indexes):
    fwd_flat = forward_indexes[:REMAIN].astype(jnp.int32).reshape(N_OUT)
    patches_flat = patches.reshape(T * B, C)
    kept = _gather(fwd_flat, patches_flat).reshape(REMAIN, B, C)
    return (kept, forward_indexes, backward_indexes)


# speedup vs baseline: 48.7614x; 1.0331x over previous
import functools

import jax
import jax.numpy as jnp
from jax import lax
from jax.experimental import pallas as pl
from jax.experimental.pallas import tpu as pltpu
from jax.experimental.pallas import tpu_sc as plsc

T, B, C = 576, 64, 768
REMAIN = T // 4
N_OUT = REMAIN * B
NW = 32
ROWS_PER_W = N_OUT // NW
CH = 48
NCHUNK = ROWS_PER_W // CH
L = 16


NBUF = 3


def _body(fwd_hbm, patches_hbm, out_hbm, fwdv, idxv,
          buf0, buf1, buf2, gsem0, gsem1, gsem2, wsem0, wsem1, wsem2):
    wid = lax.axis_index("s") * 2 + lax.axis_index("c")
    base = wid * ROWS_PER_W

    pltpu.sync_copy(fwd_hbm.at[pl.ds(base, ROWS_PER_W)], fwdv)

    lanes = lax.iota(jnp.int32, L)
    for j in range(NCHUNK):
        for i in range(CH // L):
            p0 = base + j * CH + i * L
            off = lax.rem(p0, B)
            v = fwdv[pl.ds(j * CH + i * L, L)]
            idxv[j, pl.ds(i * L, L)] = v * B + off + lanes

    bufs = (buf0, buf1, buf2)
    gsems = (gsem0, gsem1, gsem2)
    wsems = (wsem0, wsem1, wsem2)
    gathers = [None] * NCHUNK
    writes = [None] * NCHUNK

    def start_gather(j):
        gathers[j] = pltpu.async_copy(
            patches_hbm.at[idxv.at[j]], bufs[j % NBUF], gsems[j % NBUF]
        )

    for j in range(min(NBUF, NCHUNK)):
        start_gather(j)
    for j in range(NCHUNK):
        gathers[j].wait()
        writes[j] = pltpu.async_copy(
            bufs[j % NBUF], out_hbm.at[pl.ds(base + j * CH, CH)], wsems[j % NBUF]
        )
        nxt = j + NBUF
        if nxt < NCHUNK:
            writes[j].wait()
            start_gather(nxt)
    for j in range(NCHUNK - min(NBUF, NCHUNK), NCHUNK):
        writes[j].wait()


@jax.jit
def _gather(fwd_flat, patches_flat):
    mesh = plsc.VectorSubcoreMesh(core_axis_name="c", subcore_axis_name="s")
    return pl.kernel(
        _body,
        out_type=jax.ShapeDtypeStruct((N_OUT, C), jnp.float32),
        mesh=mesh,
        scratch_types=[
            pltpu.VMEM((ROWS_PER_W,), jnp.int32),
            pltpu.VMEM((NCHUNK, CH), jnp.int32),
            pltpu.VMEM((CH, C), jnp.float32),
            pltpu.VMEM((CH, C), jnp.float32),
            pltpu.VMEM((CH, C), jnp.float32),
            pltpu.SemaphoreType.DMA,
            pltpu.SemaphoreType.DMA,
            pltpu.SemaphoreType.DMA,
            pltpu.SemaphoreType.DMA,
            pltpu.SemaphoreType.DMA,
            pltpu.SemaphoreType.DMA,
        ],
    )(fwd_flat, patches_flat)


def kernel(patches, forward_indexes, backward_indexes):
    fwd_flat = forward_indexes[:REMAIN].astype(jnp.int32).reshape(N_OUT)
    patches_flat = patches.reshape(T * B, C)
    kept = _gather(fwd_flat, patches_flat).reshape(REMAIN, B, C)
    return (kept, forward_indexes, backward_indexes)


# speedup vs baseline: 49.0774x; 1.0065x over previous
import functools

import jax
import jax.numpy as jnp
from jax import lax
from jax.experimental import pallas as pl
from jax.experimental.pallas import tpu as pltpu
from jax.experimental.pallas import tpu_sc as plsc

T, B, C = 576, 64, 768
REMAIN = T // 4
N_OUT = REMAIN * B
NW = 32
ROWS_PER_W = N_OUT // NW
CH = 48
NCHUNK = ROWS_PER_W // CH
L = 16


NBUF = 3


def _body(fwd_hbm, patches_hbm, out_hbm, fwdv, idxv,
          buf0, buf1, buf2, gsem0, gsem1, gsem2, wsem0, wsem1, wsem2):
    wid = lax.axis_index("s") * 2 + lax.axis_index("c")
    base = wid * ROWS_PER_W

    pltpu.sync_copy(fwd_hbm.at[pl.ds(base, ROWS_PER_W)], fwdv)

    lanes = lax.iota(jnp.int32, L)
    for j in range(NCHUNK):
        for i in range(CH // L):
            p0 = base + j * CH + i * L
            off = lax.rem(p0, B)
            v = fwdv[pl.ds(j * CH + i * L, L)]
            idxv[j, pl.ds(i * L, L)] = v * B + off + lanes

    bufs = (buf0, buf1, buf2)
    gsems = (gsem0, gsem1, gsem2)
    wsems = (wsem0, wsem1, wsem2)
    gathers = [None] * NCHUNK
    writes = [None] * NCHUNK

    def start_gather(j):
        gathers[j] = pltpu.async_copy(
            patches_hbm.at[idxv.at[j]], bufs[j % NBUF], gsems[j % NBUF]
        )

    for j in range(min(NBUF, NCHUNK)):
        start_gather(j)
    for j in range(NCHUNK):
        gathers[j].wait()
        writes[j] = pltpu.async_copy(
            bufs[j % NBUF], out_hbm.at[pl.ds(base + j * CH, CH)], wsems[j % NBUF]
        )
        nxt = j + NBUF
        if nxt < NCHUNK:
            writes[j].wait()
            start_gather(nxt)
    for j in range(NCHUNK - min(NBUF, NCHUNK), NCHUNK):
        writes[j].wait()


@jax.jit
def _gather(fwd_flat, patches_flat):
    mesh = plsc.VectorSubcoreMesh(core_axis_name="c", subcore_axis_name="s")
    return pl.kernel(
        _body,
        out_type=jax.ShapeDtypeStruct((N_OUT, C), jnp.float32),
        mesh=mesh,
        scratch_types=[
            pltpu.VMEM((ROWS_PER_W,), jnp.int32),
            pltpu.VMEM((NCHUNK, CH), jnp.int32),
            pltpu.VMEM((CH, C), jnp.float32),
            pltpu.VMEM((CH, C), jnp.float32),
            pltpu.VMEM((CH, C), jnp.float32),
            pltpu.SemaphoreType.DMA,
            pltpu.SemaphoreType.DMA,
            pltpu.SemaphoreType.DMA,
            pltpu.SemaphoreType.DMA,
            pltpu.SemaphoreType.DMA,
            pltpu.SemaphoreType.DMA,
        ],
    )(fwd_flat, patches_flat)


def kernel(patches, forward_indexes, backward_indexes):
    fwd_flat = forward_indexes.astype(jnp.int32).reshape(T * B)
    patches_flat = patches.reshape(T * B, C)
    kept = _gather(fwd_flat, patches_flat).reshape(REMAIN, B, C)
    return (kept, forward_indexes, backward_indexes)
